# Initial kernel scaffold; baseline (speedup 1.0000x reference)
import jax
import jax.numpy as jnp
from jax.experimental import pallas as pl


def kernel(x, weights):
    raise NotImplementedError("write your pallas kernel here")


# speedup vs baseline: 1.8619x; 1.8619x over previous
import functools

import jax
import jax.numpy as jnp
from jax import lax
from jax.experimental import pallas as pl
from jax.experimental.pallas import tpu as pltpu
from jax.experimental.pallas import tpu_sc as plsc

_INFO = plsc.get_sparse_core_info()
_NC, _NS = _INFO.num_cores, _INFO.num_subcores
_NW = _NC * _NS

_CHUNK = 8
_NBUF = 2


@functools.partial(jax.jit, static_argnums=(2, 3, 4))
def _gather_rows(idx, weights, n_tokens, d, n_chunks):
    mesh = plsc.VectorSubcoreMesh(core_axis_name="c", subcore_axis_name="s")

    @functools.partial(
        pl.kernel,
        mesh=mesh,
        out_type=jax.ShapeDtypeStruct((n_tokens, d), jnp.float32),
        scratch_types=[
            pltpu.VMEM((n_chunks, _CHUNK), jnp.int32),
            pltpu.VMEM((_NBUF, _CHUNK, d), jnp.float32),
        ] + [pltpu.SemaphoreType.DMA] * _NBUF,
    )
    def body(idx_hbm, table_hbm, out_hbm, idx_v, rows_v, *gsems):
        wid = lax.axis_index("s") * _NC + lax.axis_index("c")
        base = wid * (n_chunks * _CHUNK)

        pltpu.sync_copy(idx_hbm.at[wid], idx_v)

        def gather_start(c, b):
            pltpu.async_copy(table_hbm.at[idx_v.at[c]], rows_v.at[b], gsems[b])

        def gather_wait(b):
            pltpu.make_async_copy(
                table_hbm.at[idx_v.at[0]], rows_v.at[b], gsems[b]
            ).wait()

        for b in range(_NBUF):
            gather_start(b, b)

        def group(g, carry):
            for b in range(_NBUF):
                c = g * _NBUF + b
                gather_wait(b)
                pltpu.sync_copy(rows_v.at[b], out_hbm.at[pl.ds(base + c * _CHUNK, _CHUNK)])
                nxt = c + _NBUF

                @pl.when(nxt < n_chunks)
                def _():
                    gather_start(nxt, b)
            return carry

        lax.fori_loop(0, n_chunks // _NBUF, group, 0)

    return body(idx, weights)


def kernel(x, weights):
    bt, s = x.shape
    v, d = weights.shape
    n = bt * s
    per_w = n // _NW
    n_chunks = per_w // _CHUNK
    idx = x.reshape(_NW, n_chunks, _CHUNK).astype(jnp.int32)
    out = _gather_rows(idx, weights, n, d, n_chunks)
    return out.reshape(bt, s, d)


# speedup vs baseline: 1.8767x; 1.0079x over previous
import functools

import jax
import jax.numpy as jnp
from jax import lax
from jax.experimental import pallas as pl
from jax.experimental.pallas import tpu as pltpu
from jax.experimental.pallas import tpu_sc as plsc

_INFO = plsc.get_sparse_core_info()
_NC, _NS = _INFO.num_cores, _INFO.num_subcores
_NW = _NC * _NS

_CHUNK = 8
_NBUF = 3


@functools.partial(jax.jit, static_argnums=(2, 3, 4))
def _gather_rows(idx, weights, n_tokens, d, n_chunks):
    mesh = plsc.VectorSubcoreMesh(core_axis_name="c", subcore_axis_name="s")

    @functools.partial(
        pl.kernel,
        mesh=mesh,
        out_type=jax.ShapeDtypeStruct((n_tokens, d), jnp.float32),
        scratch_types=[
            pltpu.VMEM((n_chunks, _CHUNK), jnp.int32),
            pltpu.VMEM((_NBUF, _CHUNK, d), jnp.float32),
        ] + [pltpu.SemaphoreType.DMA] * _NBUF,
    )
    def body(idx_hbm, table_hbm, out_hbm, idx_v, rows_v, *gsems):
        wid = lax.axis_index("s") * _NC + lax.axis_index("c")
        base = wid * (n_chunks * _CHUNK)

        pltpu.sync_copy(idx_hbm.at[wid], idx_v)

        def gather_start(c, b):
            pltpu.async_copy(table_hbm.at[idx_v.at[c]], rows_v.at[b], gsems[b])

        def gather_wait(b):
            pltpu.make_async_copy(
                table_hbm.at[idx_v.at[0]], rows_v.at[b], gsems[b]
            ).wait()

        for b in range(_NBUF):
            gather_start(b, b)

        def step(c, b):
            gather_wait(b)
            pltpu.sync_copy(rows_v.at[b], out_hbm.at[pl.ds(base + c * _CHUNK, _CHUNK)])

        def group(g, carry):
            for b in range(_NBUF):
                c = g * _NBUF + b
                step(c, b)
                nxt = c + _NBUF

                @pl.when(nxt < n_chunks)
                def _():
                    gather_start(nxt, b)
            return carry

        main = (n_chunks // _NBUF) * _NBUF
        lax.fori_loop(0, n_chunks // _NBUF, group, 0)
        for c in range(main, n_chunks):
            step(c, c % _NBUF)

    return body(idx, weights)


def kernel(x, weights):
    bt, s = x.shape
    v, d = weights.shape
    n = bt * s
    per_w = n // _NW
    n_chunks = per_w // _CHUNK
    idx = x.reshape(_NW, n_chunks, _CHUNK).astype(jnp.int32)
    out = _gather_rows(idx, weights, n, d, n_chunks)
    return out.reshape(bt, s, d)
